# baseline (speedup 1.0000x reference)
import jax
import jax.numpy as jnp
from jax.experimental import pallas as pl

_BLK = 8


def _diag_kernel(e_ref, out_ref):
    vals = jnp.exp(e_ref[...])
    i = jax.lax.broadcasted_iota(jnp.int32, (_BLK, 128, 128), 1)
    j = jax.lax.broadcasted_iota(jnp.int32, (_BLK, 128, 128), 2)
    out_ref[...] = jnp.where(i == j, vals, jnp.float32(0))


def kernel(embeddings):
    batch, dim, _ = embeddings.shape
    return pl.pallas_call(
        _diag_kernel,
        grid=(batch // _BLK,),
        in_specs=[pl.BlockSpec((_BLK, dim, 1), lambda b: (b, 0, 0))],
        out_specs=pl.BlockSpec((_BLK, dim, dim), lambda b: (b, 0, 0)),
        out_shape=jax.ShapeDtypeStruct((batch, dim, dim), embeddings.dtype),
    )(embeddings)


# speedup vs baseline: 3.8178x; 3.8178x over previous
import jax
import jax.numpy as jnp
from jax.experimental import pallas as pl

_BLK = 32


def _diag_kernel(e_ref, out_ref):
    dim = e_ref.shape[1]
    vals = jnp.exp(e_ref[...])
    i = jax.lax.broadcasted_iota(jnp.int32, (dim, dim), 0)
    j = jax.lax.broadcasted_iota(jnp.int32, (dim, dim), 1)
    eye = jnp.where(i == j, jnp.float32(1), jnp.float32(0))
    out_ref[...] = vals[:, :, None] * eye[None, :, :]


def kernel(embeddings):
    batch, dim, _ = embeddings.shape
    e2 = embeddings[:, :, 0]
    return pl.pallas_call(
        _diag_kernel,
        grid=(batch // _BLK,),
        in_specs=[pl.BlockSpec((_BLK, dim), lambda b: (b, 0))],
        out_specs=pl.BlockSpec((_BLK, dim, dim), lambda b: (b, 0, 0)),
        out_shape=jax.ShapeDtypeStruct((batch, dim, dim), embeddings.dtype),
    )(e2)


# speedup vs baseline: 4.8510x; 1.2706x over previous
import jax
import jax.numpy as jnp
from jax.experimental import pallas as pl

_BLK = 64


def _diag_kernel(e_ref, out_ref):
    dim = e_ref.shape[1]
    vals = jnp.exp(e_ref[...])
    i = jax.lax.broadcasted_iota(jnp.int32, (dim, dim), 0)
    j = jax.lax.broadcasted_iota(jnp.int32, (dim, dim), 1)
    eye = jnp.where(i == j, jnp.float32(1), jnp.float32(0))
    out_ref[...] = vals[:, :, None] * eye[None, :, :]


def kernel(embeddings):
    batch, dim, _ = embeddings.shape
    e2 = embeddings[:, :, 0]
    return pl.pallas_call(
        _diag_kernel,
        grid=(batch // _BLK,),
        in_specs=[pl.BlockSpec((_BLK, dim), lambda b: (b, 0))],
        out_specs=pl.BlockSpec((_BLK, dim, dim), lambda b: (b, 0, 0)),
        out_shape=jax.ShapeDtypeStruct((batch, dim, dim), embeddings.dtype),
    )(e2)


# speedup vs baseline: 5.2387x; 1.0799x over previous
import jax
import jax.numpy as jnp
from jax.experimental import pallas as pl

_BLK = 128


def _diag_kernel(e_ref, out_ref):
    dim = e_ref.shape[1]
    vals = jnp.exp(e_ref[...])
    i = jax.lax.broadcasted_iota(jnp.int32, (dim, dim), 0)
    j = jax.lax.broadcasted_iota(jnp.int32, (dim, dim), 1)
    eye = jnp.where(i == j, jnp.float32(1), jnp.float32(0))
    out_ref[...] = vals[:, :, None] * eye[None, :, :]


def kernel(embeddings):
    batch, dim, _ = embeddings.shape
    e2 = embeddings[:, :, 0]
    return pl.pallas_call(
        _diag_kernel,
        grid=(batch // _BLK,),
        in_specs=[pl.BlockSpec((_BLK, dim), lambda b: (b, 0))],
        out_specs=pl.BlockSpec((_BLK, dim, dim), lambda b: (b, 0, 0)),
        out_shape=jax.ShapeDtypeStruct((batch, dim, dim), embeddings.dtype),
    )(e2)
